# Initial kernel scaffold; baseline (speedup 1.0000x reference)
import jax
import jax.numpy as jnp
from jax.experimental import pallas as pl


def kernel(x, edge_index, W1, b1, W2, b2, Wfc, bfc):
    raise NotImplementedError("write your pallas kernel here")


# speedup vs baseline: 90.6272x; 90.6272x over previous
import jax
import jax.numpy as jnp
from jax import lax
from jax.experimental import pallas as pl
from jax.experimental.pallas import tpu as pltpu

N = 118
E = 372
EP = 384
TB = 8


def _gcn_body(x_ref, ei_ref, W1_ref, b1_ref, W2_ref, Wfc_ref, bias_ref,
              out_ref, A_scr, M_scr):
    @pl.when(pl.program_id(0) == 0)
    def _build_adjacency():
        src = ei_ref[:, 0:1]
        dst = ei_ref[:, 1:2]
        col = lax.broadcasted_iota(jnp.int32, (EP, N), 1)
        row = lax.broadcasted_iota(jnp.int32, (EP, N), 0)
        valid = row < E
        oh_src = jnp.where((src == col) & valid, 1.0, 0.0)
        oh_dst = jnp.where((dst == col) & valid, 1.0, 0.0)
        deg = jnp.sum(oh_dst, axis=0, keepdims=True) + 1.0
        dis = lax.rsqrt(deg)
        dis_s = jnp.sum(oh_src * dis, axis=1, keepdims=True)
        dis_d = jnp.sum(oh_dst * dis, axis=1, keepdims=True)
        norm = dis_s * dis_d
        A_edges = lax.dot_general(oh_dst * norm, oh_src,
                                  (((0,), (0,)), ((), ())))
        r = lax.broadcasted_iota(jnp.int32, (N, N), 0)
        c = lax.broadcasted_iota(jnp.int32, (N, N), 1)
        A = A_edges + jnp.where(r == c, dis * dis, 0.0)
        A_scr[...] = A
        M_scr[...] = lax.dot_general(Wfc_ref[...], A,
                                     (((1,), (0,)), ((), ())))

    A = A_scr[...]
    M = M_scr[...]
    W1 = W1_ref[...]
    W2 = W2_ref[...]
    b1 = b1_ref[...]
    bias = bias_ref[...]
    for b in range(TB):
        xb = x_ref[b]
        Y = lax.dot_general(W1, xb, (((0,), (0,)), ((), ())))
        Z = lax.dot_general(Y, A, (((1,), (1,)), ((), ())))
        R = jnp.maximum(Z + b1, 0.0)
        T = lax.dot_general(W2, R, (((0,), (0,)), ((), ())))
        U = lax.dot_general(M, T, (((1,), (1,)), ((), ())))
        out_ref[b] = U + bias


def kernel(x, edge_index, W1, b1, W2, b2, Wfc, bfc):
    B, F, _ = x.shape
    ei_t = jnp.zeros((EP, 2), jnp.int32).at[:E].set(edge_index.T.astype(jnp.int32))
    b1c = b1.reshape(W1.shape[1], 1)
    bias_out = (jnp.sum(Wfc, axis=1)[:, None] * b2[None, :]
                + bfc[:, None]).astype(jnp.float32)
    grid = (B // TB,)
    out = pl.pallas_call(
        _gcn_body,
        grid=grid,
        in_specs=[
            pl.BlockSpec((TB, F, N), lambda i: (i, 0, 0)),
            pl.BlockSpec((EP, 2), lambda i: (0, 0)),
            pl.BlockSpec(W1.shape, lambda i: (0, 0)),
            pl.BlockSpec((W1.shape[1], 1), lambda i: (0, 0)),
            pl.BlockSpec(W2.shape, lambda i: (0, 0)),
            pl.BlockSpec(Wfc.shape, lambda i: (0, 0)),
            pl.BlockSpec((54, 24), lambda i: (0, 0)),
        ],
        out_specs=pl.BlockSpec((TB, 54, 24), lambda i: (i, 0, 0)),
        out_shape=jax.ShapeDtypeStruct((B, 54, 24), jnp.float32),
        scratch_shapes=[
            pltpu.VMEM((N, N), jnp.float32),
            pltpu.VMEM((54, N), jnp.float32),
        ],
        compiler_params=pltpu.CompilerParams(
            dimension_semantics=("arbitrary",)),
    )(x, ei_t, W1, b1c, W2, Wfc, bias_out)
    return out
